# Initial kernel scaffold; baseline (speedup 1.0000x reference)
import functools

import jax
import jax.numpy as jnp
from jax import lax
from jax.experimental import pallas as pl
from jax.experimental.pallas import tpu as pltpu
from jax.experimental.pallas import tpu_sc as plsc

_EMB = 128
_DIMS = (81, 8, 12, 12, 10, 6, 2)
_OFFS = (0, 81, 89, 101, 113, 123, 129)
_VTOT = 131
_NC = 2
_NS = 16
_NW = _NC * _NS
_CH = 224


def _body(wcat_hbm, xt_hbm, out_hbm, tbl_v, idx_v, out_v, bpw):
    cid = lax.axis_index("c")
    sid = lax.axis_index("s")
    wid = sid * _NC + cid
    base = wid * bpw

    pltpu.sync_copy(wcat_hbm, tbl_v)
    for i in range(7):
        pltpu.sync_copy(xt_hbm.at[i, pl.ds(base, bpw)], idx_v.at[i])

    col0 = lax.iota(jnp.int32, 16)
    nchunks = bpw // _CH

    def chunk_body(ci, _):
        c0 = ci * _CH

        def row_body(j, _):
            r = c0 + j
            rvec = jnp.full((16,), r, jnp.int32)
            splats = [
                plsc.load_gather(idx_v, [jnp.full((16,), i, jnp.int32), rvec])
                for i in range(7)
            ]
            for cc in range(8):
                cols = col0 + (16 * cc)
                v = plsc.load_gather(tbl_v, [splats[0], cols])
                for i in range(1, 7):
                    v = v + plsc.load_gather(tbl_v, [splats[i], cols])
                out_v[j, pl.ds(16 * cc, 16)] = v
            return 0

        lax.fori_loop(0, _CH, row_body, 0)
        pltpu.sync_copy(out_v, out_hbm.at[pl.ds(base + c0, _CH)])
        return 0

    lax.fori_loop(0, nchunks, chunk_body, 0)


def kernel(x, W0, W1, W2, W3, W4, W5, W6):
    n = x.shape[0]
    bpw = ((n + _NW * _CH - 1) // (_NW * _CH)) * _CH
    npad = bpw * _NW

    wcat = jnp.concatenate([W0, W1, W2, W3, W4, W5, W6], axis=0)
    offs = jnp.array(_OFFS, dtype=jnp.int32)
    xt = jnp.transpose(x + offs[None, :])
    xt = jnp.pad(xt, ((0, 0), (0, npad - n)))

    mesh = plsc.VectorSubcoreMesh(core_axis_name="c", subcore_axis_name="s")
    fn = pl.kernel(
        functools.partial(_body, bpw=bpw),
        out_type=jax.ShapeDtypeStruct((npad, _EMB), jnp.float32),
        mesh=mesh,
        scratch_types=[
            pltpu.VMEM((_VTOT, _EMB), jnp.float32),
            pltpu.VMEM((7, bpw), jnp.int32),
            pltpu.VMEM((_CH, _EMB), jnp.float32),
        ],
    )
    res = fn(wcat, xt)
    return res[:n]


# speedup vs baseline: 3.6987x; 3.6987x over previous
import functools

import jax
import jax.numpy as jnp
from jax import lax
from jax.experimental import pallas as pl
from jax.experimental.pallas import tpu as pltpu
from jax.experimental.pallas import tpu_sc as plsc

_EMB = 128
_DIMS = (81, 8, 12, 12, 10, 6, 2)
_OFFS = (0, 81, 89, 101, 113, 123, 129)
_VTOT = 131
_VPAD = 136
_NC = 2
_NS = 16
_NW = _NC * _NS
_CH = 224


def _body(wcat_hbm, xt_hbm, out_hbm, tbl_v, idx_v, out_v, bpw):
    cid = lax.axis_index("c")
    sid = lax.axis_index("s")
    wid = sid * _NC + cid
    base = wid * bpw

    npad = bpw * _NW
    pltpu.sync_copy(wcat_hbm, tbl_v)
    for i in range(7):
        pltpu.sync_copy(
            xt_hbm.at[pl.ds(i * npad + base, bpw)], idx_v.at[pl.ds(i * bpw, bpw)]
        )

    col0 = lax.iota(jnp.int32, 16)
    nchunks = bpw // _CH

    def chunk_body(ci, _):
        c0 = ci * _CH

        def row_body(j, _):
            r = c0 + j
            rvec = jnp.full((16,), r, jnp.int32)
            splats = [
                plsc.load_gather(idx_v, [jnp.full((16,), i * bpw, jnp.int32) + rvec])
                for i in range(7)
            ]
            for cc in range(8):
                cols = col0 + (16 * cc)
                v = plsc.load_gather(tbl_v, [splats[0], cols])
                for i in range(1, 7):
                    v = v + plsc.load_gather(tbl_v, [splats[i], cols])
                out_v[j, pl.ds(16 * cc, 16)] = v
            return 0

        lax.fori_loop(0, _CH, row_body, 0)
        pltpu.sync_copy(out_v, out_hbm.at[pl.ds(base + c0, _CH)])
        return 0

    lax.fori_loop(0, nchunks, chunk_body, 0)


def kernel(x, W0, W1, W2, W3, W4, W5, W6):
    n = x.shape[0]
    bpw = ((n + _NW * _CH - 1) // (_NW * _CH)) * _CH
    npad = bpw * _NW

    wcat = jnp.concatenate([W0, W1, W2, W3, W4, W5, W6], axis=0)
    wcat = jnp.pad(wcat, ((0, _VPAD - _VTOT), (0, 0)))
    offs = jnp.array(_OFFS, dtype=jnp.int32)
    xt = jnp.transpose(x + offs[None, :])
    xt = jnp.pad(xt, ((0, 0), (0, npad - n))).reshape(-1)

    mesh = plsc.VectorSubcoreMesh(core_axis_name="c", subcore_axis_name="s")
    fn = pl.kernel(
        functools.partial(_body, bpw=bpw),
        out_type=jax.ShapeDtypeStruct((npad, _EMB), jnp.float32),
        mesh=mesh,
        compiler_params=pltpu.CompilerParams(needs_layout_passes=False),
        scratch_types=[
            pltpu.VMEM((_VPAD, _EMB), jnp.float32),
            pltpu.VMEM((7 * bpw,), jnp.int32),
            pltpu.VMEM((_CH, _EMB), jnp.float32),
        ],
    )
    res = fn(wcat, xt)
    return res[:n]


# speedup vs baseline: 4.0456x; 1.0938x over previous
import functools

import jax
import jax.numpy as jnp
from jax import lax
from jax.experimental import pallas as pl
from jax.experimental.pallas import tpu as pltpu
from jax.experimental.pallas import tpu_sc as plsc

_EMB = 128
_DIMS = (81, 8, 12, 12, 10, 6, 2)
_VTOT = 131
_VPAD = 136
_NCOMB = 232
_NC = 2
_NS = 16
_NW = _NC * _NS
_CH = 224


def _splat(val):
    return jnp.full((16,), val, jnp.int32)


def _body(wcat_hbm, xt_hbm, out_hbm, tbl_v, comb_v, idxc_v, cidx_v, out_v, bpw):
    cid = lax.axis_index("c")
    sid = lax.axis_index("s")
    wid = sid * _NC + cid
    base = wid * bpw
    npad = bpw * _NW

    pltpu.sync_copy(wcat_hbm, tbl_v)

    col0 = lax.iota(jnp.int32, 16)
    cols = [col0 + 16 * cc for cc in range(8)]

    def build(dst_off, src1_off, d1, src2_off, d2):
        def outer(a, _):
            sa = _splat(src1_off + a)

            def inner(b, _):
                sb = _splat(src2_off + b)
                r = dst_off + a * d2 + b
                for cc in range(8):
                    v = plsc.load_gather(tbl_v, [sa, cols[cc]])
                    v = v + plsc.load_gather(tbl_v, [sb, cols[cc]])
                    comb_v[r, pl.ds(16 * cc, 16)] = v
                return 0

            lax.fori_loop(0, d2, inner, 0)
            return 0

        lax.fori_loop(0, d1, outer, 0)

    build(0, 81, 8, 89, 12)
    build(96, 101, 12, 113, 10)
    build(216, 123, 6, 129, 2)

    nchunks = bpw // _CH

    def chunk_body(ci, _):
        c0 = ci * _CH
        for i in range(7):
            pltpu.sync_copy(
                xt_hbm.at[pl.ds(i * npad + base + c0, _CH)],
                idxc_v.at[pl.ds(i * _CH, _CH)],
            )

        def idx_body(g, _):
            g0 = g * 16
            xs = [idxc_v[pl.ds(i * _CH + g0, 16)] for i in range(7)]
            cidx_v[pl.ds(g0, 16)] = xs[0]
            cidx_v[pl.ds(_CH + g0, 16)] = xs[1] * 12 + xs[2]
            cidx_v[pl.ds(2 * _CH + g0, 16)] = xs[3] * 10 + xs[4] + 96
            cidx_v[pl.ds(3 * _CH + g0, 16)] = xs[5] * 2 + xs[6] + 216
            return 0

        lax.fori_loop(0, _CH // 16, idx_body, 0)

        def row_body(j, _):
            rvec = _splat(0) + j
            sA = plsc.load_gather(cidx_v, [rvec])
            sB = plsc.load_gather(cidx_v, [rvec + _CH])
            sC = plsc.load_gather(cidx_v, [rvec + 2 * _CH])
            sD = plsc.load_gather(cidx_v, [rvec + 3 * _CH])
            for cc in range(8):
                v = plsc.load_gather(tbl_v, [sA, cols[cc]])
                v = v + plsc.load_gather(comb_v, [sB, cols[cc]])
                v = v + plsc.load_gather(comb_v, [sC, cols[cc]])
                v = v + plsc.load_gather(comb_v, [sD, cols[cc]])
                out_v[j, pl.ds(16 * cc, 16)] = v
            return 0

        lax.fori_loop(0, _CH, row_body, 0)
        pltpu.sync_copy(out_v, out_hbm.at[pl.ds(base + c0, _CH)])
        return 0

    lax.fori_loop(0, nchunks, chunk_body, 0)


def kernel(x, W0, W1, W2, W3, W4, W5, W6):
    n = x.shape[0]
    bpw = ((n + _NW * _CH - 1) // (_NW * _CH)) * _CH
    npad = bpw * _NW

    wcat = jnp.concatenate([W0, W1, W2, W3, W4, W5, W6], axis=0)
    wcat = jnp.pad(wcat, ((0, _VPAD - _VTOT), (0, 0)))
    xt = jnp.transpose(x)
    xt = jnp.pad(xt, ((0, 0), (0, npad - n))).reshape(-1)

    mesh = plsc.VectorSubcoreMesh(core_axis_name="c", subcore_axis_name="s")
    fn = pl.kernel(
        functools.partial(_body, bpw=bpw),
        out_type=jax.ShapeDtypeStruct((npad, _EMB), jnp.float32),
        mesh=mesh,
        compiler_params=pltpu.CompilerParams(needs_layout_passes=False),
        scratch_types=[
            pltpu.VMEM((_VPAD, _EMB), jnp.float32),
            pltpu.VMEM((_NCOMB, _EMB), jnp.float32),
            pltpu.VMEM((7 * _CH,), jnp.int32),
            pltpu.VMEM((4 * _CH,), jnp.int32),
            pltpu.VMEM((_CH, _EMB), jnp.float32),
        ],
    )
    res = fn(wcat, xt)
    return res[:n]


# speedup vs baseline: 7.1671x; 1.7716x over previous
import functools

import jax
import jax.numpy as jnp
from jax import lax
from jax.experimental import pallas as pl
from jax.experimental.pallas import tpu as pltpu
from jax.experimental.pallas import tpu_sc as plsc

_EMB = 128
_DIMS = (81, 8, 12, 12, 10, 6, 2)
_VTOT = 131
_VPAD = 136
_NCOMB = 232
_NC = 2
_NS = 16
_NW = _NC * _NS
_CH = 224


def _splat(val):
    return jnp.full((16,), val, jnp.int32)


def _body(wcat_hbm, xt_hbm, out_hbm, tbl_v, comb_v, idxc_v, cidx_v, out_v, bpw):
    cid = lax.axis_index("c")
    sid = lax.axis_index("s")
    wid = sid * _NC + cid
    base = wid * bpw
    npad = bpw * _NW

    pltpu.sync_copy(wcat_hbm, tbl_v)

    col0 = lax.iota(jnp.int32, 16)
    cols = [col0 + 16 * cc for cc in range(8)]

    def build(dst_off, src1_off, d1, src2_off, d2):
        def outer(a, _):
            sa = _splat(src1_off + a)

            def inner(b, _):
                sb = _splat(src2_off + b)
                r = dst_off + a * d2 + b
                for cc in range(8):
                    v = plsc.load_gather(tbl_v, [sa, cols[cc]])
                    v = v + plsc.load_gather(tbl_v, [sb, cols[cc]])
                    comb_v[r, pl.ds(16 * cc, 16)] = v
                return 0

            lax.fori_loop(0, d2, inner, 0)
            return 0

        lax.fori_loop(0, d1, outer, 0)

    build(0, 81, 8, 89, 12)
    build(96, 101, 12, 113, 10)
    build(216, 123, 6, 129, 2)

    nchunks = bpw // _CH

    def chunk_body(ci, _):
        c0 = ci * _CH
        for i in range(7):
            pltpu.sync_copy(
                xt_hbm.at[pl.ds(i * npad + base + c0, _CH)],
                idxc_v.at[pl.ds(i * _CH, _CH)],
            )

        @plsc.parallel_loop(0, _CH // 16, unroll=2)
        def idx_body(g):
            g0 = g * 16
            xs = [idxc_v[pl.ds(i * _CH + g0, 16)] for i in range(7)]
            cidx_v[pl.ds(g0, 16)] = xs[0]
            cidx_v[pl.ds(_CH + g0, 16)] = xs[1] * 12 + xs[2]
            cidx_v[pl.ds(2 * _CH + g0, 16)] = xs[3] * 10 + xs[4] + 96
            cidx_v[pl.ds(3 * _CH + g0, 16)] = xs[5] * 2 + xs[6] + 216

        @plsc.parallel_loop(0, _CH, unroll=4)
        def row_body(j):
            rvec = _splat(0) + j
            sA = plsc.load_gather(cidx_v, [rvec])
            sB = plsc.load_gather(cidx_v, [rvec + _CH])
            sC = plsc.load_gather(cidx_v, [rvec + 2 * _CH])
            sD = plsc.load_gather(cidx_v, [rvec + 3 * _CH])
            for cc in range(8):
                vab = plsc.load_gather(tbl_v, [sA, cols[cc]]) + plsc.load_gather(
                    comb_v, [sB, cols[cc]]
                )
                vcd = plsc.load_gather(comb_v, [sC, cols[cc]]) + plsc.load_gather(
                    comb_v, [sD, cols[cc]]
                )
                out_v[j, pl.ds(16 * cc, 16)] = vab + vcd
        pltpu.sync_copy(out_v, out_hbm.at[pl.ds(base + c0, _CH)])
        return 0

    lax.fori_loop(0, nchunks, chunk_body, 0)


def kernel(x, W0, W1, W2, W3, W4, W5, W6):
    n = x.shape[0]
    bpw = ((n + _NW * _CH - 1) // (_NW * _CH)) * _CH
    npad = bpw * _NW

    wcat = jnp.concatenate([W0, W1, W2, W3, W4, W5, W6], axis=0)
    wcat = jnp.pad(wcat, ((0, _VPAD - _VTOT), (0, 0)))
    xt = jnp.transpose(x)
    xt = jnp.pad(xt, ((0, 0), (0, npad - n))).reshape(-1)

    mesh = plsc.VectorSubcoreMesh(core_axis_name="c", subcore_axis_name="s")
    fn = pl.kernel(
        functools.partial(_body, bpw=bpw),
        out_type=jax.ShapeDtypeStruct((npad, _EMB), jnp.float32),
        mesh=mesh,
        compiler_params=pltpu.CompilerParams(needs_layout_passes=False),
        scratch_types=[
            pltpu.VMEM((_VPAD, _EMB), jnp.float32),
            pltpu.VMEM((_NCOMB, _EMB), jnp.float32),
            pltpu.VMEM((7 * _CH,), jnp.int32),
            pltpu.VMEM((4 * _CH,), jnp.int32),
            pltpu.VMEM((_CH, _EMB), jnp.float32),
        ],
    )
    res = fn(wcat, xt)
    return res[:n]


# speedup vs baseline: 7.1784x; 1.0016x over previous
import functools

import jax
import jax.numpy as jnp
import numpy as np
from jax import lax
from jax.experimental import pallas as pl
from jax.experimental.pallas import tpu as pltpu
from jax.experimental.pallas import tpu_sc as plsc

_EMB = 128
_DIMS = (81, 8, 12, 12, 10, 6, 2)
_VTOT = 131
_VPAD = 136
_NCOMB = 232
_NC = 2
_NS = 16
_NW = _NC * _NS
_CH = 224
_BLK = 7 * _CH


def _splat(val):
    return jnp.full((16,), val, jnp.int32)


def _body(wcat_hbm, xr_hbm, out_hbm, tbl_v, comb_v, idxc_v, cidx_v,
          out_v0, out_v1, isem, osem, n, bpw, nch):
    cid = lax.axis_index("c")
    sid = lax.axis_index("s")
    wid = sid * _NC + cid
    base = wid * bpw

    pltpu.async_copy(
        xr_hbm.at[pl.ds(wid * nch * _BLK, _BLK)], idxc_v.at[pl.ds(0, _BLK)], isem
    )
    pltpu.sync_copy(wcat_hbm, tbl_v)

    col0 = lax.iota(jnp.int32, 16)
    cols = [col0 + 16 * cc for cc in range(8)]

    def build(dst_off, src1_off, d1, src2_off, d2):
        def outer(a, _):
            sa = _splat(src1_off + a)

            @plsc.parallel_loop(0, d2, unroll=2)
            def inner(b):
                sb = _splat(src2_off + b)
                r = dst_off + a * d2 + b
                for cc in range(8):
                    v = plsc.load_gather(tbl_v, [sa, cols[cc]])
                    v = v + plsc.load_gather(tbl_v, [sb, cols[cc]])
                    comb_v[r, pl.ds(16 * cc, 16)] = v

            return 0

        lax.fori_loop(0, d1, outer, 0)

    build(0, 81, 8, 89, 12)
    build(96, 101, 12, 113, 10)
    build(216, 123, 6, 129, 2)

    def do_chunk(ci, ioff, noff, out_buf):
        pltpu.make_async_copy(
            xr_hbm.at[pl.ds(0, _BLK)], idxc_v.at[pl.ds(ioff, _BLK)], isem
        ).wait()

        @pl.when(ci + 1 < nch)
        def _():
            pltpu.async_copy(
                xr_hbm.at[pl.ds((wid * nch + ci + 1) * _BLK, _BLK)],
                idxc_v.at[pl.ds(noff, _BLK)], isem,
            )

        @pl.when(ci >= 2)
        def _():
            pltpu.make_async_copy(
                out_hbm.at[pl.ds(0, _CH)], out_buf, osem
            ).wait()

        @plsc.parallel_loop(0, _CH // 16, unroll=2)
        def idx_body(g):
            g0 = g * 16
            xs = [idxc_v[pl.ds(ioff + i * _CH + g0, 16)] for i in range(7)]
            cidx_v[pl.ds(g0, 16)] = xs[0]
            cidx_v[pl.ds(_CH + g0, 16)] = xs[1] * 12 + xs[2]
            cidx_v[pl.ds(2 * _CH + g0, 16)] = xs[3] * 10 + xs[4] + 96
            cidx_v[pl.ds(3 * _CH + g0, 16)] = xs[5] * 2 + xs[6] + 216

        @plsc.parallel_loop(0, _CH, unroll=6)
        def row_body(j):
            rvec = _splat(0) + j
            sA = plsc.load_gather(cidx_v, [rvec])
            sB = plsc.load_gather(cidx_v, [rvec + _CH])
            sC = plsc.load_gather(cidx_v, [rvec + 2 * _CH])
            sD = plsc.load_gather(cidx_v, [rvec + 3 * _CH])
            for cc in range(8):
                vab = plsc.load_gather(tbl_v, [sA, cols[cc]]) + plsc.load_gather(
                    comb_v, [sB, cols[cc]]
                )
                vcd = plsc.load_gather(comb_v, [sC, cols[cc]]) + plsc.load_gather(
                    comb_v, [sD, cols[cc]]
                )
                out_buf[j, pl.ds(16 * cc, 16)] = vab + vcd

        gstart = jnp.minimum(base + ci * _CH, n - _CH)
        pltpu.async_copy(out_buf, out_hbm.at[pl.ds(gstart, _CH)], osem)

    def chunk_pair(ci2, _):
        do_chunk(2 * ci2, 0, _BLK, out_v0)
        do_chunk(2 * ci2 + 1, _BLK, 0, out_v1)
        return 0

    lax.fori_loop(0, nch // 2, chunk_pair, 0)

    for _ in range(2):
        pltpu.make_async_copy(out_hbm.at[pl.ds(0, _CH)], out_v0, osem).wait()


def kernel(x, W0, W1, W2, W3, W4, W5, W6):
    n = x.shape[0]
    bpw = ((n + _NW * _CH - 1) // (_NW * _CH)) * _CH
    nch = bpw // _CH

    wcat = jnp.concatenate([W0, W1, W2, W3, W4, W5, W6], axis=0)
    wcat = jnp.pad(wcat, ((0, _VPAD - _VTOT), (0, 0)))
    starts = np.minimum(
        (np.arange(_NW)[:, None] * bpw + np.arange(nch)[None, :] * _CH), n - _CH
    )
    rows = starts.reshape(-1, 1) + np.arange(_CH)[None, :]
    xr = jnp.transpose(x[jnp.asarray(rows.reshape(-1))].reshape(-1, _CH, 7),
                       (0, 2, 1)).reshape(-1)

    mesh = plsc.VectorSubcoreMesh(core_axis_name="c", subcore_axis_name="s")
    fn = pl.kernel(
        functools.partial(_body, n=n, bpw=bpw, nch=nch),
        out_type=jax.ShapeDtypeStruct((n, _EMB), jnp.float32),
        mesh=mesh,
        compiler_params=pltpu.CompilerParams(needs_layout_passes=False),
        scratch_types=[
            pltpu.VMEM((_VPAD, _EMB), jnp.float32),
            pltpu.VMEM((_NCOMB, _EMB), jnp.float32),
            pltpu.VMEM((2 * _BLK,), jnp.int32),
            pltpu.VMEM((4 * _CH,), jnp.int32),
            pltpu.VMEM((_CH, _EMB), jnp.float32),
            pltpu.VMEM((_CH, _EMB), jnp.float32),
            pltpu.SemaphoreType.DMA,
            pltpu.SemaphoreType.DMA,
        ],
    )
    return fn(wcat, xr)


# speedup vs baseline: 8.7509x; 1.2191x over previous
import functools

import jax
import jax.numpy as jnp
import numpy as np
from jax import lax
from jax.experimental import pallas as pl
from jax.experimental.pallas import tpu as pltpu
from jax.experimental.pallas import tpu_sc as plsc

_EMB = 128
_DIMS = (81, 8, 12, 12, 10, 6, 2)
_VTOT = 131
_VPAD = 136
_NCOMB = 232
_NC = 2
_NS = 16
_NW = _NC * _NS
_CH = 224
_BLK = 7 * _CH


def _splat(val):
    return jnp.full((16,), val, jnp.int32)


def _body(wcat_hbm, xr_hbm, out_hbm, tbl_v, comb_v, idxc_v, cidx_v,
          out_v0, out_v1, isem, osem, n, bpw, nch):
    cid = lax.axis_index("c")
    sid = lax.axis_index("s")
    wid = sid * _NC + cid
    base = wid * bpw

    pltpu.async_copy(
        xr_hbm.at[pl.ds(base * 7, _BLK)], idxc_v.at[pl.ds(0, _BLK)], isem
    )
    pltpu.sync_copy(wcat_hbm, tbl_v)

    col0 = lax.iota(jnp.int32, 16)
    cols = [col0 + 16 * cc for cc in range(8)]

    def build(dst_off, src1_off, d1, src2_off, d2):
        def outer(a, _):
            sa = _splat(src1_off + a)

            @plsc.parallel_loop(0, d2, unroll=2)
            def inner(b):
                sb = _splat(src2_off + b)
                r = dst_off + a * d2 + b
                for cc in range(8):
                    v = plsc.load_gather(tbl_v, [sa, cols[cc]])
                    v = v + plsc.load_gather(tbl_v, [sb, cols[cc]])
                    comb_v[r, pl.ds(16 * cc, 16)] = v

            return 0

        lax.fori_loop(0, d1, outer, 0)

    build(0, 81, 8, 89, 12)
    build(96, 101, 12, 113, 10)
    build(216, 123, 6, 129, 2)

    col7 = col0 * 7

    def do_chunk(ci, ioff, noff, out_buf):
        pltpu.make_async_copy(
            xr_hbm.at[pl.ds(0, _BLK)], idxc_v.at[pl.ds(ioff, _BLK)], isem
        ).wait()

        @pl.when(ci + 1 < nch)
        def _():
            nstart = jnp.minimum(base + (ci + 1) * _CH, n - _CH)
            pltpu.async_copy(
                xr_hbm.at[pl.ds(nstart * 7, _BLK)],
                idxc_v.at[pl.ds(noff, _BLK)], isem,
            )

        @pl.when(ci >= 2)
        def _():
            pltpu.make_async_copy(
                out_hbm.at[pl.ds(0, _CH)], out_buf, osem
            ).wait()

        @plsc.parallel_loop(0, _CH // 16, unroll=2)
        def idx_body(g):
            g0 = g * 16
            xs = [
                plsc.load_gather(idxc_v, [col7 + (ioff + g0 * 7 + i)])
                for i in range(7)
            ]
            cidx_v[pl.ds(g0, 16)] = xs[0]
            cidx_v[pl.ds(_CH + g0, 16)] = xs[1] * 12 + xs[2]
            cidx_v[pl.ds(2 * _CH + g0, 16)] = xs[3] * 10 + xs[4] + 96
            cidx_v[pl.ds(3 * _CH + g0, 16)] = xs[5] * 2 + xs[6] + 216

        @plsc.parallel_loop(0, _CH, unroll=6)
        def row_body(j):
            rvec = _splat(0) + j
            sA = plsc.load_gather(cidx_v, [rvec])
            sB = plsc.load_gather(cidx_v, [rvec + _CH])
            sC = plsc.load_gather(cidx_v, [rvec + 2 * _CH])
            sD = plsc.load_gather(cidx_v, [rvec + 3 * _CH])
            for cc in range(8):
                vab = plsc.load_gather(tbl_v, [sA, cols[cc]]) + plsc.load_gather(
                    comb_v, [sB, cols[cc]]
                )
                vcd = plsc.load_gather(comb_v, [sC, cols[cc]]) + plsc.load_gather(
                    comb_v, [sD, cols[cc]]
                )
                out_buf[j, pl.ds(16 * cc, 16)] = vab + vcd

        gstart = jnp.minimum(base + ci * _CH, n - _CH)
        pltpu.async_copy(out_buf, out_hbm.at[pl.ds(gstart, _CH)], osem)

    def chunk_pair(ci2, _):
        do_chunk(2 * ci2, 0, _BLK, out_v0)
        do_chunk(2 * ci2 + 1, _BLK, 0, out_v1)
        return 0

    lax.fori_loop(0, nch // 2, chunk_pair, 0)

    for _ in range(2):
        pltpu.make_async_copy(out_hbm.at[pl.ds(0, _CH)], out_v0, osem).wait()


def kernel(x, W0, W1, W2, W3, W4, W5, W6):
    n = x.shape[0]
    bpw = ((n + _NW * _CH - 1) // (_NW * _CH)) * _CH
    nch = bpw // _CH

    wcat = jnp.concatenate([W0, W1, W2, W3, W4, W5, W6], axis=0)
    wcat = jnp.pad(wcat, ((0, _VPAD - _VTOT), (0, 0)))
    xr = x.reshape(-1)

    mesh = plsc.VectorSubcoreMesh(core_axis_name="c", subcore_axis_name="s")
    fn = pl.kernel(
        functools.partial(_body, n=n, bpw=bpw, nch=nch),
        out_type=jax.ShapeDtypeStruct((n, _EMB), jnp.float32),
        mesh=mesh,
        compiler_params=pltpu.CompilerParams(needs_layout_passes=False),
        scratch_types=[
            pltpu.VMEM((_VPAD, _EMB), jnp.float32),
            pltpu.VMEM((_NCOMB, _EMB), jnp.float32),
            pltpu.VMEM((2 * _BLK,), jnp.int32),
            pltpu.VMEM((4 * _CH,), jnp.int32),
            pltpu.VMEM((_CH, _EMB), jnp.float32),
            pltpu.VMEM((_CH, _EMB), jnp.float32),
            pltpu.SemaphoreType.DMA,
            pltpu.SemaphoreType.DMA,
        ],
    )
    return fn(wcat, xr)
